# baseline (speedup 1.0000x reference)
import functools

import jax
import jax.numpy as jnp
from jax import lax
from jax.experimental import pallas as pl
from jax.experimental.pallas import tpu as pltpu
from jax.experimental.pallas import tpu_sc as plsc

B = 16384
D = 128
NAGE = 120
NGND = 4
NCOMBO = NAGE * NGND

NC = 2
NS = 16
NW = NC * NS
BPW = B // NW
CH = 128
NCH = BPW // CH
LANES = 16
TPS = NCOMBO // NS
DK = D // LANES

_RSQRT_MAGIC = 0x5F3759DF


def _hsum(v):
    lanes = lax.iota(jnp.int32, LANES)
    dnums = lax.GatherDimensionNumbers(
        offset_dims=(), collapsed_slice_dims=(0,), start_index_map=(0,))
    for sh in (8, 4, 2, 1):
        idx = (lanes + sh) & (LANES - 1)
        v = v + lax.gather(v, idx[:, None], dnums, slice_sizes=(1,),
                           mode=lax.GatherScatterMode.PROMISE_IN_BOUNDS)
    return v


def _vec_rsqrt(v):
    y = lax.bitcast_convert_type(
        _RSQRT_MAGIC - (lax.bitcast_convert_type(v, jnp.int32) >> 1), jnp.float32)
    half = 0.5 * v
    for _ in range(3):
        y = y * (1.5 - half * y * y)
    return y


def _sc_body(age_hbm, gnd_hbm, age_t_hbm, gnd_t_hbm, gamma_hbm, beta_hbm,
             out_hbm,
             cidx, av, gv, rows, atv, gtv, gbv, tslab, tspm,
             g0, g1, g2, g3, ia, ig, it, ssem):
    gsems = (g0, g1, g2, g3)
    sid = lax.axis_index("s")
    wid = sid * NC + lax.axis_index("c")
    base = wid * BPW

    age_cp = pltpu.async_copy(age_hbm.at[pl.ds(base, BPW)], av, ia)
    gnd_cp = pltpu.async_copy(gnd_hbm.at[pl.ds(base, BPW)], gv, ig)
    g = sid // (NAGE // TPS)
    a0 = (sid % (NAGE // TPS)) * TPS
    at_cp = pltpu.async_copy(age_t_hbm.at[pl.ds(a0 * 64, TPS * 64)], atv, it)
    gt_cp = pltpu.async_copy(gnd_t_hbm.at[pl.ds(g * 64, 64)], gtv, g0)
    gm_cp = pltpu.async_copy(gamma_hbm, gbv.at[0], g1)
    bt_cp = pltpu.async_copy(beta_hbm, gbv.at[1], g2)
    at_cp.wait()
    gt_cp.wait()
    gm_cp.wait()
    bt_cp.wait()

    gvec = [gbv[0, pl.ds(k * LANES, LANES)] for k in range(DK)]
    bvec = [gbv[1, pl.ds(k * LANES, LANES)] for k in range(DK)]
    gnd_part = [gtv[pl.ds(k * LANES, LANES)] for k in range(DK // 2)]
    for j in range(TPS):
        x = [atv[pl.ds(j * 64 + k * LANES, LANES)] for k in range(DK // 2)] + gnd_part
        s = x[0]
        for k in range(1, DK):
            s = s + x[k]
        mean = _hsum(s) * (1.0 / D)
        c = [xk - mean for xk in x]
        sq = c[0] * c[0]
        for k in range(1, DK):
            sq = sq + c[k] * c[k]
        var = _hsum(sq) * (1.0 / D)
        rstd = _vec_rsqrt(var + 1e-6)
        for k in range(DK):
            tslab[j, pl.ds(k * LANES, LANES)] = c[k] * rstd * gvec[k] + bvec[k]
    pltpu.sync_copy(tslab, tspm.at[pl.ds(sid * TPS, TPS)])

    age_cp.wait()
    gnd_cp.wait()
    for k in range(NCH):
        for i in range(CH // LANES):
            sl = pl.ds(i * LANES, LANES)
            src = pl.ds(k * CH + i * LANES, LANES)
            cidx[k, sl] = gv[src] * NAGE + av[src]
    plsc.subcore_barrier()

    gathers = [
        pltpu.async_copy(tspm.at[cidx.at[k]], rows.at[k], gsems[k])
        for k in range(NCH)
    ]
    scatters = []
    for k in range(NCH):
        gathers[k].wait()
        scatters.append(
            pltpu.async_copy(rows.at[k], out_hbm.at[pl.ds(base + k * CH, CH)], ssem)
        )
    for s in scatters:
        s.wait()


@functools.lru_cache(maxsize=None)
def _make_sc_kernel():
    mesh = plsc.VectorSubcoreMesh(
        core_axis_name="c", subcore_axis_name="s", num_cores=NC, num_subcores=NS
    )
    return pl.kernel(
        _sc_body,
        out_type=jax.ShapeDtypeStruct((B, D), jnp.float32),
        mesh=mesh,
        scratch_types=[
            pltpu.VMEM((NCH, CH), jnp.int32),
            pltpu.VMEM((BPW,), jnp.int32),
            pltpu.VMEM((BPW,), jnp.int32),
            pltpu.VMEM((NCH, CH, D), jnp.float32),
            pltpu.VMEM((TPS * 64,), jnp.float32),
            pltpu.VMEM((64,), jnp.float32),
            pltpu.VMEM((2, D), jnp.float32),
            pltpu.VMEM((TPS, D), jnp.float32),
            pltpu.VMEM_SHARED((NCOMBO, D), jnp.float32),
            pltpu.SemaphoreType.DMA,
            pltpu.SemaphoreType.DMA,
            pltpu.SemaphoreType.DMA,
            pltpu.SemaphoreType.DMA,
            pltpu.SemaphoreType.DMA,
            pltpu.SemaphoreType.DMA,
            pltpu.SemaphoreType.DMA,
            pltpu.SemaphoreType.DMA,
        ],
    )


def kernel(age, gnd, age_table, gnd_table, gamma, beta):
    age = age.astype(jnp.int32)
    gnd = gnd.astype(jnp.int32)
    return _make_sc_kernel()(age, gnd, age_table.reshape(-1),
                             gnd_table.reshape(-1), gamma, beta)


# speedup vs baseline: 1.1123x; 1.1123x over previous
import functools

import jax
import jax.numpy as jnp
from jax import lax
from jax.experimental import pallas as pl
from jax.experimental.pallas import tpu as pltpu
from jax.experimental.pallas import tpu_sc as plsc

B = 16384
D = 128
NAGE = 120
NGND = 4
NCOMBO = NAGE * NGND

NC = 2
NS = 16
NW = NC * NS
BPW = B // NW
CH = 128
NCH = BPW // CH
LANES = 16


def _combo_table_body(age_t_ref, gnd_t_ref, gamma_ref, beta_ref, t_ref):
    at = age_t_ref[...]
    gt = gnd_t_ref[...]
    s = (jnp.sum(at, axis=1, keepdims=True)[None, :, :]
         + jnp.sum(gt, axis=1, keepdims=True)[:, None, :])
    mean = s / D
    ca = at[None, :, :] - mean
    cg = gt[:, None, :] - mean
    var = (jnp.sum(ca * ca, axis=2, keepdims=True)
           + jnp.sum(cg * cg, axis=2, keepdims=True)) / D
    rstd = lax.rsqrt(var + 1e-6)
    gamma = gamma_ref[...]
    beta = beta_ref[...]
    left = ca * rstd * gamma[None, :, :64] + beta[None, :, :64]
    right = cg * rstd * gamma[None, :, 64:] + beta[None, :, 64:]
    t_ref[...] = jnp.concatenate([left, right], axis=-1)


def _build_combo_table(age_table, gnd_table, gamma, beta):
    t3 = pl.pallas_call(
        _combo_table_body,
        out_shape=jax.ShapeDtypeStruct((NGND, NAGE, D), jnp.float32),
    )(age_table, gnd_table, gamma.reshape(1, D), beta.reshape(1, D))
    return t3.reshape(NCOMBO, D)


def _sc_gather_body(age_hbm, gnd_hbm, t_hbm, out_hbm,
                    cidx, av, gv, rows, tspm, g0, g1, g2, g3, ia, ig, ssem):
    gsems = (g0, g1, g2, g3)
    sid = lax.axis_index("s")
    wid = sid * NC + lax.axis_index("c")
    base = wid * BPW
    age_cp = pltpu.async_copy(age_hbm.at[pl.ds(base, BPW)], av, ia)
    gnd_cp = pltpu.async_copy(gnd_hbm.at[pl.ds(base, BPW)], gv, ig)
    @pl.when(sid == 0)
    def _():
        pltpu.sync_copy(t_hbm, tspm)
    age_cp.wait()
    gnd_cp.wait()
    gathers = []
    for k in range(NCH):
        for i in range(CH // LANES):
            sl = pl.ds(i * LANES, LANES)
            src = pl.ds(k * CH + i * LANES, LANES)
            cidx[k, sl] = gv[src] * NAGE + av[src]
        if k == 0:
            plsc.subcore_barrier()
        gathers.append(
            pltpu.async_copy(tspm.at[cidx.at[k]], rows.at[k], gsems[k])
        )
    scatters = []
    for k in range(NCH):
        gathers[k].wait()
        scatters.append(
            pltpu.async_copy(rows.at[k], out_hbm.at[pl.ds(base + k * CH, CH)], ssem)
        )
    for s in scatters:
        s.wait()


@functools.lru_cache(maxsize=None)
def _make_sc_gather():
    mesh = plsc.VectorSubcoreMesh(
        core_axis_name="c", subcore_axis_name="s", num_cores=NC, num_subcores=NS
    )
    return pl.kernel(
        _sc_gather_body,
        out_type=jax.ShapeDtypeStruct((B, D), jnp.float32),
        mesh=mesh,
        scratch_types=[
            pltpu.VMEM((NCH, CH), jnp.int32),
            pltpu.VMEM((BPW,), jnp.int32),
            pltpu.VMEM((BPW,), jnp.int32),
            pltpu.VMEM((NCH, CH, D), jnp.float32),
            pltpu.VMEM_SHARED((NCOMBO, D), jnp.float32),
            pltpu.SemaphoreType.DMA,
            pltpu.SemaphoreType.DMA,
            pltpu.SemaphoreType.DMA,
            pltpu.SemaphoreType.DMA,
            pltpu.SemaphoreType.DMA,
            pltpu.SemaphoreType.DMA,
            pltpu.SemaphoreType.DMA,
        ],
    )


def kernel(age, gnd, age_table, gnd_table, gamma, beta):
    age = age.astype(jnp.int32)
    gnd = gnd.astype(jnp.int32)
    t = _build_combo_table(age_table, gnd_table, gamma, beta)
    return _make_sc_gather()(age, gnd, t)


# speedup vs baseline: 1.1234x; 1.0100x over previous
import functools

import jax
import jax.numpy as jnp
from jax import lax
from jax.experimental import pallas as pl
from jax.experimental.pallas import tpu as pltpu
from jax.experimental.pallas import tpu_sc as plsc

B = 16384
D = 128
NAGE = 120
NGND = 4
NCOMBO = NAGE * NGND

NC = 2
NS = 16
NW = NC * NS
BPW = B // NW
CH = 64
NCH = BPW // CH
LANES = 16


def _combo_table_body(age_t_ref, gnd_t_ref, gamma_ref, beta_ref, t_ref):
    at = age_t_ref[...]
    gt = gnd_t_ref[...]
    s = (jnp.sum(at, axis=1, keepdims=True)[None, :, :]
         + jnp.sum(gt, axis=1, keepdims=True)[:, None, :])
    mean = s / D
    ca = at[None, :, :] - mean
    cg = gt[:, None, :] - mean
    var = (jnp.sum(ca * ca, axis=2, keepdims=True)
           + jnp.sum(cg * cg, axis=2, keepdims=True)) / D
    rstd = lax.rsqrt(var + 1e-6)
    gamma = gamma_ref[...]
    beta = beta_ref[...]
    left = ca * rstd * gamma[None, :, :64] + beta[None, :, :64]
    right = cg * rstd * gamma[None, :, 64:] + beta[None, :, 64:]
    t_ref[...] = jnp.concatenate([left, right], axis=-1)


def _build_combo_table(age_table, gnd_table, gamma, beta):
    t3 = pl.pallas_call(
        _combo_table_body,
        out_shape=jax.ShapeDtypeStruct((NGND, NAGE, D), jnp.float32),
    )(age_table, gnd_table, gamma.reshape(1, D), beta.reshape(1, D))
    return t3.reshape(NCOMBO, D)


def _sc_gather_body(age_hbm, gnd_hbm, t_hbm, out_hbm,
                    cidx, av, gv, rows, tspm,
                    g0, g1, g2, g3, g4, g5, g6, g7, ia, ig, it, ssem):
    gsems = (g0, g1, g2, g3, g4, g5, g6, g7)
    sid = lax.axis_index("s")
    wid = sid * NC + lax.axis_index("c")
    base = wid * BPW
    age_cp = pltpu.async_copy(age_hbm.at[pl.ds(base, BPW)], av, ia)
    gnd_cp = pltpu.async_copy(gnd_hbm.at[pl.ds(base, BPW)], gv, ig)
    t_cp = None
    @pl.when(sid == 0)
    def _():
        nonlocal t_cp
        t_cp = pltpu.async_copy(t_hbm, tspm, it)
    age_cp.wait()
    gnd_cp.wait()
    gathers = []
    for k in range(NCH):
        for i in range(CH // LANES):
            sl = pl.ds(i * LANES, LANES)
            src = pl.ds(k * CH + i * LANES, LANES)
            cidx[k, sl] = gv[src] * NAGE + av[src]
        if k == 0:
            @pl.when(sid == 0)
            def _():
                t_cp.wait()
            plsc.subcore_barrier()
        gathers.append(
            pltpu.async_copy(tspm.at[cidx.at[k]], rows.at[k], gsems[k])
        )
    scatters = []
    for k in range(NCH):
        gathers[k].wait()
        scatters.append(
            pltpu.async_copy(rows.at[k], out_hbm.at[pl.ds(base + k * CH, CH)], ssem)
        )
    for s in scatters:
        s.wait()


@functools.lru_cache(maxsize=None)
def _make_sc_gather():
    mesh = plsc.VectorSubcoreMesh(
        core_axis_name="c", subcore_axis_name="s", num_cores=NC, num_subcores=NS
    )
    return pl.kernel(
        _sc_gather_body,
        out_type=jax.ShapeDtypeStruct((B, D), jnp.float32),
        mesh=mesh,
        scratch_types=[
            pltpu.VMEM((NCH, CH), jnp.int32),
            pltpu.VMEM((BPW,), jnp.int32),
            pltpu.VMEM((BPW,), jnp.int32),
            pltpu.VMEM((NCH, CH, D), jnp.float32),
            pltpu.VMEM_SHARED((NCOMBO, D), jnp.float32),
            pltpu.SemaphoreType.DMA,
            pltpu.SemaphoreType.DMA,
            pltpu.SemaphoreType.DMA,
            pltpu.SemaphoreType.DMA,
            pltpu.SemaphoreType.DMA,
            pltpu.SemaphoreType.DMA,
            pltpu.SemaphoreType.DMA,
            pltpu.SemaphoreType.DMA,
            pltpu.SemaphoreType.DMA,
            pltpu.SemaphoreType.DMA,
            pltpu.SemaphoreType.DMA,
            pltpu.SemaphoreType.DMA,
        ],
    )


def kernel(age, gnd, age_table, gnd_table, gamma, beta):
    age = age.astype(jnp.int32)
    gnd = gnd.astype(jnp.int32)
    t = _build_combo_table(age_table, gnd_table, gamma, beta)
    return _make_sc_gather()(age, gnd, t)
